# Initial kernel scaffold; baseline (speedup 1.0000x reference)
import jax
import jax.numpy as jnp
from jax.experimental import pallas as pl


def kernel(quer_feat_lc, quer_feat_te, lc_W1, lc_b1, lc_W2, lc_b2, te_W1, te_b1, te_W2, te_b2, bin_score):
    raise NotImplementedError("write your pallas kernel here")


# speedup vs baseline: 2.8535x; 2.8535x over previous
import jax
import jax.numpy as jnp
from jax import lax
from jax.experimental import pallas as pl
from jax.experimental.pallas import tpu as pltpu

D_MODEL = 256
SINK_ITERS = 50
N = 1000
PAD = 1024


def _ot_kernel(xlc_ref, xte_ref, w1l_ref, b1l_ref, w2l_ref, b2l_ref,
               w1t_ref, b1t_ref, w2t_ref, b2t_ref, alpha_ref, out_ref):
    f32 = jnp.float32
    xlc = xlc_ref[0]
    xte = xte_ref[0]

    h = jnp.maximum(jnp.dot(xlc, w1l_ref[...], preferred_element_type=f32)
                    + b1l_ref[...], 0.0)
    f_lc = jnp.dot(h, w2l_ref[...], preferred_element_type=f32) + b2l_ref[...]
    h = jnp.maximum(jnp.dot(xte, w1t_ref[...], preferred_element_type=f32)
                    + b1t_ref[...], 0.0)
    f_te = jnp.dot(h, w2t_ref[...], preferred_element_type=f32) + b2t_ref[...]

    inv_sqrt_d = 1.0 / (D_MODEL ** 0.5)
    s = lax.dot_general(f_lc, f_te, (((1,), (1,)), ((), ())),
                        preferred_element_type=f32) * inv_sqrt_d
    st = lax.dot_general(f_te, f_lc, (((1,), (1,)), ((), ())),
                         preferred_element_type=f32) * inv_sqrt_d

    alpha = alpha_ref[0, 0]
    ri = lax.broadcasted_iota(jnp.int32, (PAD, PAD), 0)
    ci = lax.broadcasted_iota(jnp.int32, (PAD, PAD), 1)
    in_scores = (ri < N) & (ci < N)
    in_coupl = (ri <= N) & (ci <= N)

    C = jnp.where(in_scores, s, alpha)
    K = jnp.where(in_coupl, jnp.exp(C), 0.0)
    CT = jnp.where(in_scores, st, alpha)
    KT = jnp.where(in_coupl, jnp.exp(CT), 0.0)

    rcol = lax.broadcasted_iota(jnp.int32, (PAD, 1), 0)
    mu = jnp.where(rcol < N, 1.0 / (2.0 * N),
                   jnp.where(rcol == N, 0.5, 0.0)).astype(f32)
    live = rcol <= N

    def body(_, uv):
        pu, pv = uv
        r = jnp.dot(K, pv, preferred_element_type=f32)
        pu = mu / jnp.where(live, r, 1.0)
        c = jnp.dot(KT, pu, preferred_element_type=f32)
        pv = mu / jnp.where(live, c, 1.0)
        return (pu, pv)

    pu0 = jnp.ones((PAD, 1), f32)
    pu, pv = lax.fori_loop(0, SINK_ITERS, body, (pu0, pu0))

    u = jnp.log(pu)
    vt = jnp.log(pv).reshape(1, PAD)
    out_ref[0] = C + u + vt + jnp.log(2.0 * N).astype(f32)


def kernel(quer_feat_lc, quer_feat_te, lc_W1, lc_b1, lc_W2, lc_b2,
           te_W1, te_b1, te_W2, te_b2, bin_score):
    B = quer_feat_lc.shape[0]
    pad_rows = PAD - quer_feat_lc.shape[1]
    xlc = jnp.pad(quer_feat_lc, ((0, 0), (0, pad_rows), (0, 0)))
    xte = jnp.pad(quer_feat_te, ((0, 0), (0, pad_rows), (0, 0)))
    alpha = jnp.reshape(bin_score, (1, 1)).astype(jnp.float32)

    d = D_MODEL
    mat = pl.BlockSpec((d, d), lambda b: (0, 0))
    vec = pl.BlockSpec((1, d), lambda b: (0, 0))

    out = pl.pallas_call(
        _ot_kernel,
        grid=(B,),
        in_specs=[
            pl.BlockSpec((1, PAD, d), lambda b: (b, 0, 0)),
            pl.BlockSpec((1, PAD, d), lambda b: (b, 0, 0)),
            mat, vec, mat, vec,
            mat, vec, mat, vec,
            pl.BlockSpec((1, 1), lambda b: (0, 0)),
        ],
        out_specs=pl.BlockSpec((1, PAD, PAD), lambda b: (b, 0, 0)),
        out_shape=jax.ShapeDtypeStruct((B, PAD, PAD), jnp.float32),
        compiler_params=pltpu.CompilerParams(
            dimension_semantics=("parallel",)),
    )(xlc, xte,
      lc_W1, lc_b1.reshape(1, d), lc_W2, lc_b2.reshape(1, d),
      te_W1, te_b1.reshape(1, d), te_W2, te_b2.reshape(1, d),
      alpha)
    return out[:, :N + 1, :N + 1]
